# baseline (speedup 1.0000x reference)
import functools

import jax
import jax.numpy as jnp
from jax import lax
from jax.experimental import pallas as pl
from jax.experimental.pallas import tpu as pltpu
from jax.experimental.pallas import tpu_sc as plsc

NC = 2
NS = 16
L = 16
NW = NC * NS

C = 8
BSUB = 128
G = BSUB // L
CB = 8192

_CP = pltpu.CompilerParams(needs_layout_passes=False, use_tc_tiling_on_sc=False)
_MESH = dict(core_axis_name="c", subcore_axis_name="s")


def _make_tc_transpose(HW, HpW):
    grid = (HpW + CB - 1) // CB
    jmax = (HW - 1) // CB

    def body(in_ref, out_ref):
        j = pl.program_id(0)
        t = in_ref[...]
        ir = lax.broadcasted_iota(jnp.int32, (C, C), 0)
        ic = lax.broadcasted_iota(jnp.int32, (C, C), 1)
        eye = jnp.where(ir == ic, 1.0, 0.0).astype(jnp.float32)
        tt = lax.dot_general(t, eye, (((0,), (0,)), ((), ())),
                             precision=lax.Precision.HIGHEST,
                             preferred_element_type=jnp.float32)
        cid = lax.broadcasted_iota(jnp.int32, (CB, C), 0) + j * CB
        out_ref[...] = jnp.where(cid < HW, tt, 0.0)

    return pl.pallas_call(
        body,
        grid=(grid,),
        in_specs=[pl.BlockSpec((C, CB), lambda j: (0, jnp.minimum(j, jmax)))],
        out_specs=pl.BlockSpec((CB, C), lambda j: (j, 0)),
        out_shape=jax.ShapeDtypeStruct((HpW, C), jnp.float32),
    )


def _full(v):
    return jnp.full((L,), v, jnp.int32)


def _dual(r, half):
    return jnp.right_shift(r, 1) + jnp.bitwise_and(r, 1) * half


def _bilerp(ref0, ref1, pi, cv, cv8, wx, wy):
    v00 = plsc.load_gather(ref0, [pi, cv])
    v10 = plsc.load_gather(ref0, [pi, cv8])
    v01 = plsc.load_gather(ref1, [pi, cv])
    v11 = plsc.load_gather(ref1, [pi, cv8])
    a = v00 + wx * (v10 - v00)
    b = v01 + wx * (v11 - v01)
    return a + wy * (b - a)


def _make_sc_kernel(N, Wyx, Hyx, Wzx, Hzx, Wzy, Hzy, half_a, half_b, half_c):
    BT = N // NW
    NSUB = BT // BSUB
    mesh = plsc.VectorSubcoreMesh(**_MESH)
    f32 = jnp.float32
    i32 = jnp.int32

    @functools.partial(
        pl.kernel,
        mesh=mesh,
        out_type=jax.ShapeDtypeStruct((N, 3), f32),
        compiler_params=_CP,
        scratch_types=[
            pltpu.VMEM((BSUB,), f32),
            pltpu.VMEM((BSUB,), f32),
            pltpu.VMEM((BSUB,), f32),
            pltpu.VMEM((BSUB,), i32),
            pltpu.VMEM((BSUB,), i32),
            pltpu.VMEM((BSUB,), i32),
            pltpu.VMEM((BSUB,), i32),
            pltpu.VMEM((BSUB,), i32),
            pltpu.VMEM((BSUB,), i32),
            pltpu.VMEM((BSUB,), f32),
            pltpu.VMEM((BSUB,), f32),
            pltpu.VMEM((BSUB,), f32),
            pltpu.VMEM((BSUB, 2 * C), f32),
            pltpu.VMEM((BSUB, 2 * C), f32),
            pltpu.VMEM((BSUB, 2 * C), f32),
            pltpu.VMEM((BSUB, 2 * C), f32),
            pltpu.VMEM((BSUB, 2 * C), f32),
            pltpu.VMEM((BSUB, 2 * C), f32),
            pltpu.VMEM((32,), f32),
            pltpu.VMEM((BSUB, 3), f32),
            pltpu.SemaphoreType.DMA,
        ],
    )
    def sc_kernel(xs_h, ys_h, zs_h, tyx_h, tzx_h, tzy_h, wb_h, out_h,
                  xs_v, ys_v, zs_v,
                  i0a_v, i1a_v, i0b_v, i1b_v, i0c_v, i1c_v,
                  wx_v, wy_v, wz_v,
                  r0a_v, r1a_v, r0b_v, r1b_v, r0c_v, r1c_v,
                  wb_v, out_v, sem):
        wid = lax.axis_index("s") * NC + lax.axis_index("c")
        pltpu.sync_copy(wb_h, wb_v)
        wsp = [[plsc.load_gather(wb_v, [_full(1 + cc * 3 + j)]) for j in range(3)]
               for cc in range(C)]
        bsp = [plsc.load_gather(wb_v, [_full(1 + C * 3 + j)]) for j in range(3)]

        def sub(s, carry):
            base = wid * BT + s * BSUB
            pltpu.sync_copy(xs_h.at[pl.ds(base, BSUB)], xs_v)
            pltpu.sync_copy(ys_h.at[pl.ds(base, BSUB)], ys_v)
            pltpu.sync_copy(zs_h.at[pl.ds(base, BSUB)], zs_v)
            for g in range(G):
                sl = pl.ds(g * L, L)
                x = xs_v[sl]
                y = ys_v[sl]
                z = zs_v[sl]
                ax = (x + 1.0) * 0.5 * (Wyx - 1)
                ay = (y + 1.0) * 0.5 * (Hyx - 1)
                az = (z + 1.0) * 0.5 * (Hzx - 1)
                xi = ax.astype(i32)
                yi = ay.astype(i32)
                zi = az.astype(i32)
                wx_v[sl] = ax - xi.astype(f32)
                wy_v[sl] = ay - yi.astype(f32)
                wz_v[sl] = az - zi.astype(f32)
                r0 = yi * Wyx + xi
                i0a_v[sl] = _dual(r0, half_a)
                i1a_v[sl] = _dual(r0 + Wyx, half_a)
                r0 = zi * Wzx + xi
                i0b_v[sl] = _dual(r0, half_b)
                i1b_v[sl] = _dual(r0 + Wzx, half_b)
                r0 = zi * Wzy + yi
                i0c_v[sl] = _dual(r0, half_c)
                i1c_v[sl] = _dual(r0 + Wzy, half_c)
            cps = [
                pltpu.async_copy(tyx_h.at[i0a_v], r0a_v, sem),
                pltpu.async_copy(tyx_h.at[i1a_v], r1a_v, sem),
                pltpu.async_copy(tzx_h.at[i0b_v], r0b_v, sem),
                pltpu.async_copy(tzx_h.at[i1b_v], r1b_v, sem),
                pltpu.async_copy(tzy_h.at[i0c_v], r0c_v, sem),
                pltpu.async_copy(tzy_h.at[i1c_v], r1c_v, sem),
            ]
            for cp in cps:
                cp.wait()
            for g in range(G):
                sl = pl.ds(g * L, L)
                pi = lax.iota(i32, L) + g * L
                wx = wx_v[sl]
                wy = wy_v[sl]
                wz = wz_v[sl]
                acc = [bsp[0], bsp[1], bsp[2]]
                for cc in range(C):
                    cv = _full(cc)
                    cv8 = _full(cc + C)
                    fa = _bilerp(r0a_v, r1a_v, pi, cv, cv8, wx, wy)
                    fb = _bilerp(r0b_v, r1b_v, pi, cv, cv8, wx, wz)
                    fc = _bilerp(r0c_v, r1c_v, pi, cv, cv8, wy, wz)
                    f = fa * fb * fc
                    for j in range(3):
                        acc[j] = acc[j] + f * wsp[cc][j]
                for j in range(3):
                    plsc.store_scatter(out_v, [pi, _full(j)], acc[j])
            pltpu.sync_copy(out_v, out_h.at[pl.ds(base, BSUB)])
            return carry

        lax.fori_loop(0, NSUB, sub, 0)

    return sc_kernel


def _build_table(plane):
    c, H, W = plane.shape
    HpW = (H + 2) * W
    HpW += HpW % 2
    f = _make_tc_transpose(H * W, HpW)(plane.reshape(c, H * W))
    r2 = HpW // 2 - 1
    even = f[:2 * r2].reshape(r2, 2 * C)
    odd = f[1:2 * r2 + 1].reshape(r2, 2 * C)
    return jnp.concatenate([even, odd], axis=0), r2


def kernel(pts, plane_yx, plane_zx, plane_zy, W_dec, b_dec):
    N = pts.shape[0]
    r2s = []
    ts = []
    for p in (plane_yx, plane_zx, plane_zy):
        t, r2 = _build_table(p)
        ts.append(t)
        r2s.append(r2)
    tyx, tzx, tzy = ts
    _, Hyx, Wyx = plane_yx.shape
    _, Hzx, Wzx = plane_zx.shape
    _, Hzy, Wzy = plane_zy.shape
    xs = pts[:, 0]
    ys = pts[:, 1]
    zs = pts[:, 2]
    wb = jnp.concatenate(
        [jnp.zeros((1,), jnp.float32), W_dec.reshape(-1), b_dec,
         jnp.zeros((4,), jnp.float32)], axis=0)
    sc = _make_sc_kernel(N, Wyx, Hyx, Wzx, Hzx, Wzy, Hzy, *r2s)
    return sc(xs, ys, zs, tyx, tzx, tzy, wb)


# speedup vs baseline: 1.9160x; 1.9160x over previous
import functools

import jax
import jax.numpy as jnp
from jax import lax
from jax.experimental import pallas as pl
from jax.experimental.pallas import tpu as pltpu
from jax.experimental.pallas import tpu_sc as plsc

NC = 2
NS = 16
L = 16
NW = NC * NS

C = 8
BSUB = 128
G = BSUB // L

_CP = pltpu.CompilerParams(needs_layout_passes=False, use_tc_tiling_on_sc=False)
_MESH = dict(core_axis_name="c", subcore_axis_name="s")


def _full(v):
    return jnp.full((L,), v, jnp.int32)


def _dual(r, half):
    return jnp.right_shift(r, 1) + jnp.bitwise_and(r, 1) * half


def _bilerp(ref0, ref1, pi, cv, cv8, wx, wy):
    v00 = plsc.load_gather(ref0, [pi, cv])
    v10 = plsc.load_gather(ref0, [pi, cv8])
    v01 = plsc.load_gather(ref1, [pi, cv])
    v11 = plsc.load_gather(ref1, [pi, cv8])
    a = v00 + wx * (v10 - v00)
    b = v01 + wx * (v11 - v01)
    return a + wy * (b - a)


def _make_sc_kernel(N, Wyx, Hyx, Wzx, Hzx, Wzy, Hzy, half_a, half_b, half_c):
    BT = N // NW
    NSUB = BT // BSUB
    mesh = plsc.VectorSubcoreMesh(**_MESH)
    f32 = jnp.float32
    i32 = jnp.int32

    @functools.partial(
        pl.kernel,
        mesh=mesh,
        out_type=jax.ShapeDtypeStruct((N, 3), f32),
        compiler_params=_CP,
        scratch_types=[
            pltpu.VMEM((BSUB,), f32),
            pltpu.VMEM((BSUB,), f32),
            pltpu.VMEM((BSUB,), f32),
            pltpu.VMEM((BSUB,), i32),
            pltpu.VMEM((BSUB,), i32),
            pltpu.VMEM((BSUB,), i32),
            pltpu.VMEM((BSUB,), i32),
            pltpu.VMEM((BSUB,), i32),
            pltpu.VMEM((BSUB,), i32),
            pltpu.VMEM((BSUB,), f32),
            pltpu.VMEM((BSUB,), f32),
            pltpu.VMEM((BSUB,), f32),
            pltpu.VMEM((BSUB, 2 * C), f32),
            pltpu.VMEM((BSUB, 2 * C), f32),
            pltpu.VMEM((BSUB, 2 * C), f32),
            pltpu.VMEM((BSUB, 2 * C), f32),
            pltpu.VMEM((BSUB, 2 * C), f32),
            pltpu.VMEM((BSUB, 2 * C), f32),
            pltpu.VMEM((32,), f32),
            pltpu.VMEM((BSUB, 3), f32),
            pltpu.SemaphoreType.DMA,
        ],
    )
    def sc_kernel(xs_h, ys_h, zs_h, tyx_h, tzx_h, tzy_h, wb_h, out_h,
                  xs_v, ys_v, zs_v,
                  i0a_v, i1a_v, i0b_v, i1b_v, i0c_v, i1c_v,
                  wx_v, wy_v, wz_v,
                  r0a_v, r1a_v, r0b_v, r1b_v, r0c_v, r1c_v,
                  wb_v, out_v, sem):
        wid = lax.axis_index("s") * NC + lax.axis_index("c")
        pltpu.sync_copy(wb_h, wb_v)
        wsp = [[plsc.load_gather(wb_v, [_full(1 + cc * 3 + j)]) for j in range(3)]
               for cc in range(C)]
        bsp = [plsc.load_gather(wb_v, [_full(1 + C * 3 + j)]) for j in range(3)]

        def sub(s, carry):
            base = wid * BT + s * BSUB
            pcs = [
                pltpu.async_copy(xs_h.at[pl.ds(base, BSUB)], xs_v, sem),
                pltpu.async_copy(ys_h.at[pl.ds(base, BSUB)], ys_v, sem),
                pltpu.async_copy(zs_h.at[pl.ds(base, BSUB)], zs_v, sem),
            ]
            for cp in pcs:
                cp.wait()
            for g in range(G):
                sl = pl.ds(g * L, L)
                x = xs_v[sl]
                y = ys_v[sl]
                z = zs_v[sl]
                ax = (x + 1.0) * 0.5 * (Wyx - 1)
                ay = (y + 1.0) * 0.5 * (Hyx - 1)
                az = (z + 1.0) * 0.5 * (Hzx - 1)
                xi = ax.astype(i32)
                yi = ay.astype(i32)
                zi = az.astype(i32)
                wx_v[sl] = ax - xi.astype(f32)
                wy_v[sl] = ay - yi.astype(f32)
                wz_v[sl] = az - zi.astype(f32)
                r0 = yi * Wyx + xi
                i0a_v[sl] = _dual(r0, half_a)
                i1a_v[sl] = _dual(r0 + Wyx, half_a)
                r0 = zi * Wzx + xi
                i0b_v[sl] = _dual(r0, half_b)
                i1b_v[sl] = _dual(r0 + Wzx, half_b)
                r0 = zi * Wzy + yi
                i0c_v[sl] = _dual(r0, half_c)
                i1c_v[sl] = _dual(r0 + Wzy, half_c)
            cps = [
                pltpu.async_copy(tyx_h.at[i0a_v], r0a_v, sem),
                pltpu.async_copy(tyx_h.at[i1a_v], r1a_v, sem),
                pltpu.async_copy(tzx_h.at[i0b_v], r0b_v, sem),
                pltpu.async_copy(tzx_h.at[i1b_v], r1b_v, sem),
                pltpu.async_copy(tzy_h.at[i0c_v], r0c_v, sem),
                pltpu.async_copy(tzy_h.at[i1c_v], r1c_v, sem),
            ]
            for cp in cps:
                cp.wait()
            for g in range(G):
                sl = pl.ds(g * L, L)
                pi = lax.iota(i32, L) + g * L
                wx = wx_v[sl]
                wy = wy_v[sl]
                wz = wz_v[sl]
                acc = [bsp[0], bsp[1], bsp[2]]
                for cc in range(C):
                    cv = _full(cc)
                    cv8 = _full(cc + C)
                    fa = _bilerp(r0a_v, r1a_v, pi, cv, cv8, wx, wy)
                    fb = _bilerp(r0b_v, r1b_v, pi, cv, cv8, wx, wz)
                    fc = _bilerp(r0c_v, r1c_v, pi, cv, cv8, wy, wz)
                    f = fa * fb * fc
                    for j in range(3):
                        acc[j] = acc[j] + f * wsp[cc][j]
                for j in range(3):
                    plsc.store_scatter(out_v, [pi, _full(j)], acc[j])
            pltpu.sync_copy(out_v, out_h.at[pl.ds(base, BSUB)])
            return carry

        lax.fori_loop(0, NSUB, sub, 0)

    return sc_kernel


def _build_table(plane):
    c, H, W = plane.shape
    R = (H + 2) * W
    R += R % 2
    t = jnp.transpose(plane, (1, 2, 0)).reshape(-1)
    F = jnp.zeros((R * C + C,), jnp.float32).at[: H * W * c].set(t)
    even = F[: R * C].reshape(R // 2, 2 * C)
    odd = F[C: R * C + C].reshape(R // 2, 2 * C)
    return jnp.concatenate([even, odd], axis=0), R // 2


def kernel(pts, plane_yx, plane_zx, plane_zy, W_dec, b_dec):
    N = pts.shape[0]
    r2s = []
    ts = []
    for p in (plane_yx, plane_zx, plane_zy):
        t, r2 = _build_table(p)
        ts.append(t)
        r2s.append(r2)
    tyx, tzx, tzy = ts
    _, Hyx, Wyx = plane_yx.shape
    _, Hzx, Wzx = plane_zx.shape
    _, Hzy, Wzy = plane_zy.shape
    xs = pts[:, 0]
    ys = pts[:, 1]
    zs = pts[:, 2]
    wb = jnp.concatenate(
        [jnp.zeros((1,), jnp.float32), W_dec.reshape(-1), b_dec,
         jnp.zeros((4,), jnp.float32)], axis=0)
    sc = _make_sc_kernel(N, Wyx, Hyx, Wzx, Hzx, Wzy, Hzy, *r2s)
    return sc(xs, ys, zs, tyx, tzx, tzy, wb)


# speedup vs baseline: 1.9685x; 1.0274x over previous
import functools

import jax
import jax.numpy as jnp
from jax import lax
from jax.experimental import pallas as pl
from jax.experimental.pallas import tpu as pltpu
from jax.experimental.pallas import tpu_sc as plsc

NC = 2
NS = 16
L = 16
NW = NC * NS

C = 8
BSUB = 128
G = BSUB // L

_CP = pltpu.CompilerParams(needs_layout_passes=False, use_tc_tiling_on_sc=False)
_MESH = dict(core_axis_name="c", subcore_axis_name="s")


def _full(v):
    return jnp.full((L,), v, jnp.int32)


def _dual(r, half):
    return jnp.right_shift(r, 1) + jnp.bitwise_and(r, 1) * half


def _bilerp(ref0, ref1, pi, cv, cv8, wx, wy):
    v00 = plsc.load_gather(ref0, [pi, cv])
    v10 = plsc.load_gather(ref0, [pi, cv8])
    v01 = plsc.load_gather(ref1, [pi, cv])
    v11 = plsc.load_gather(ref1, [pi, cv8])
    a = v00 + wx * (v10 - v00)
    b = v01 + wx * (v11 - v01)
    return a + wy * (b - a)


def _make_sc_kernel(N, Wyx, Hyx, Wzx, Hzx, Wzy, Hzy, half_a, half_b, half_c):
    BT = N // NW
    NSUB = BT // BSUB
    mesh = plsc.VectorSubcoreMesh(**_MESH)
    f32 = jnp.float32
    i32 = jnp.int32

    @functools.partial(
        pl.kernel,
        mesh=mesh,
        out_type=jax.ShapeDtypeStruct((N, 3), f32),
        compiler_params=_CP,
        scratch_types=[
            pltpu.VMEM((BSUB,), f32),
            pltpu.VMEM((BSUB,), f32),
            pltpu.VMEM((BSUB,), f32),
        ] + [pltpu.VMEM((BSUB,), i32) for _ in range(12)]
          + [pltpu.VMEM((BSUB,), f32) for _ in range(6)]
          + [pltpu.VMEM((BSUB, 2 * C), f32) for _ in range(12)]
          + [
            pltpu.VMEM((32,), f32),
            pltpu.VMEM((BSUB, 3), f32),
            pltpu.SemaphoreType.DMA,
            pltpu.SemaphoreType.DMA,
            pltpu.SemaphoreType.DMA,
        ],
    )
    def sc_kernel(xs_h, ys_h, zs_h, tyx_h, tzx_h, tzy_h, wb_h, out_h,
                  xs_v, ys_v, zs_v,
                  i0a0, i1a0, i0b0, i1b0, i0c0, i1c0,
                  i0a1, i1a1, i0b1, i1b1, i0c1, i1c1,
                  wx0, wy0, wz0, wx1, wy1, wz1,
                  r0a0, r1a0, r0b0, r1b0, r0c0, r1c0,
                  r0a1, r1a1, r0b1, r1b1, r0c1, r1c1,
                  wb_v, out_v, psem, gsem0, gsem1):
        idxs = ((i0a0, i1a0, i0b0, i1b0, i0c0, i1c0),
                (i0a1, i1a1, i0b1, i1b1, i0c1, i1c1))
        wts = ((wx0, wy0, wz0), (wx1, wy1, wz1))
        rows = ((r0a0, r1a0, r0b0, r1b0, r0c0, r1c0),
                (r0a1, r1a1, r0b1, r1b1, r0c1, r1c1))
        gsems = (gsem0, gsem1)
        wid = lax.axis_index("s") * NC + lax.axis_index("c")
        pltpu.sync_copy(wb_h, wb_v)
        wsp = [[plsc.load_gather(wb_v, [_full(1 + cc * 3 + j)]) for j in range(3)]
               for cc in range(C)]
        bsp = [plsc.load_gather(wb_v, [_full(1 + C * 3 + j)]) for j in range(3)]

        def load_idx(base, ph):
            i0a_v, i1a_v, i0b_v, i1b_v, i0c_v, i1c_v = idxs[ph]
            wx_v, wy_v, wz_v = wts[ph]
            pcs = [
                pltpu.async_copy(xs_h.at[pl.ds(base, BSUB)], xs_v, psem),
                pltpu.async_copy(ys_h.at[pl.ds(base, BSUB)], ys_v, psem),
                pltpu.async_copy(zs_h.at[pl.ds(base, BSUB)], zs_v, psem),
            ]
            for cp in pcs:
                cp.wait()
            for g in range(G):
                sl = pl.ds(g * L, L)
                x = xs_v[sl]
                y = ys_v[sl]
                z = zs_v[sl]
                ax = (x + 1.0) * 0.5 * (Wyx - 1)
                ay = (y + 1.0) * 0.5 * (Hyx - 1)
                az = (z + 1.0) * 0.5 * (Hzx - 1)
                xi = ax.astype(i32)
                yi = ay.astype(i32)
                zi = az.astype(i32)
                wx_v[sl] = ax - xi.astype(f32)
                wy_v[sl] = ay - yi.astype(f32)
                wz_v[sl] = az - zi.astype(f32)
                r0 = yi * Wyx + xi
                i0a_v[sl] = _dual(r0, half_a)
                i1a_v[sl] = _dual(r0 + Wyx, half_a)
                r0 = zi * Wzx + xi
                i0b_v[sl] = _dual(r0, half_b)
                i1b_v[sl] = _dual(r0 + Wzx, half_b)
                r0 = zi * Wzy + yi
                i0c_v[sl] = _dual(r0, half_c)
                i1c_v[sl] = _dual(r0 + Wzy, half_c)
            i0a_v, i1a_v, i0b_v, i1b_v, i0c_v, i1c_v = idxs[ph]
            r0a_v, r1a_v, r0b_v, r1b_v, r0c_v, r1c_v = rows[ph]
            return [
                pltpu.async_copy(tyx_h.at[i0a_v], r0a_v, gsems[ph]),
                pltpu.async_copy(tyx_h.at[i1a_v], r1a_v, gsems[ph]),
                pltpu.async_copy(tzx_h.at[i0b_v], r0b_v, gsems[ph]),
                pltpu.async_copy(tzx_h.at[i1b_v], r1b_v, gsems[ph]),
                pltpu.async_copy(tzy_h.at[i0c_v], r0c_v, gsems[ph]),
                pltpu.async_copy(tzy_h.at[i1c_v], r1c_v, gsems[ph]),
            ]

        def combine(base, ph):
            wx_v, wy_v, wz_v = wts[ph]
            r0a_v, r1a_v, r0b_v, r1b_v, r0c_v, r1c_v = rows[ph]
            for g in range(G):
                sl = pl.ds(g * L, L)
                pi = lax.iota(i32, L) + g * L
                wx = wx_v[sl]
                wy = wy_v[sl]
                wz = wz_v[sl]
                acc = [bsp[0], bsp[1], bsp[2]]
                for cc in range(C):
                    cv = _full(cc)
                    cv8 = _full(cc + C)
                    fa = _bilerp(r0a_v, r1a_v, pi, cv, cv8, wx, wy)
                    fb = _bilerp(r0b_v, r1b_v, pi, cv, cv8, wx, wz)
                    fc = _bilerp(r0c_v, r1c_v, pi, cv, cv8, wy, wz)
                    f = fa * fb * fc
                    for j in range(3):
                        acc[j] = acc[j] + f * wsp[cc][j]
                for j in range(3):
                    plsc.store_scatter(out_v, [pi, _full(j)], acc[j])
            pltpu.sync_copy(out_v, out_h.at[pl.ds(base, BSUB)])

        def sub(i, carry):
            base0 = wid * BT + (2 * i) * BSUB
            base1 = base0 + BSUB
            cps0 = load_idx(base0, 0)
            cps1 = load_idx(base1, 1)
            for cp in cps0:
                cp.wait()
            combine(base0, 0)
            for cp in cps1:
                cp.wait()
            combine(base1, 1)
            return carry

        lax.fori_loop(0, NSUB // 2, sub, 0)

    return sc_kernel


def _build_table(plane):
    c, H, W = plane.shape
    R = (H + 2) * W
    R += R % 2
    t = jnp.transpose(plane, (1, 2, 0)).reshape(-1)
    F = jnp.zeros((R * C + C,), jnp.float32).at[: H * W * c].set(t)
    even = F[: R * C].reshape(R // 2, 2 * C)
    odd = F[C: R * C + C].reshape(R // 2, 2 * C)
    return jnp.concatenate([even, odd], axis=0), R // 2


def kernel(pts, plane_yx, plane_zx, plane_zy, W_dec, b_dec):
    N = pts.shape[0]
    r2s = []
    ts = []
    for p in (plane_yx, plane_zx, plane_zy):
        t, r2 = _build_table(p)
        ts.append(t)
        r2s.append(r2)
    tyx, tzx, tzy = ts
    _, Hyx, Wyx = plane_yx.shape
    _, Hzx, Wzx = plane_zx.shape
    _, Hzy, Wzy = plane_zy.shape
    xs = pts[:, 0]
    ys = pts[:, 1]
    zs = pts[:, 2]
    wb = jnp.concatenate(
        [jnp.zeros((1,), jnp.float32), W_dec.reshape(-1), b_dec,
         jnp.zeros((4,), jnp.float32)], axis=0)
    sc = _make_sc_kernel(N, Wyx, Hyx, Wzx, Hzx, Wzy, Hzy, *r2s)
    return sc(xs, ys, zs, tyx, tzx, tzy, wb)
